# baseline (speedup 1.0000x reference)
import functools
import jax
import jax.numpy as jnp
from jax import lax
from jax.experimental import pallas as pl
from jax.experimental.pallas import tpu as pltpu
from jax.experimental.pallas import tpu_sc as plsc

_NC, _NS = 2, 16
_NW = _NC * _NS
_CH = 32


def kernel(data, partitions):
    del partitions
    rows, cols = data.shape
    rows_per_w = rows // _NW
    n_chunks = rows_per_w // _CH
    mesh = plsc.VectorSubcoreMesh(core_axis_name="c", subcore_axis_name="s")

    @functools.partial(
        pl.kernel,
        mesh=mesh,
        out_type=jax.ShapeDtypeStruct((rows, cols), data.dtype),
        scratch_types=[
            pltpu.VMEM((_CH, cols), data.dtype),
            pltpu.VMEM((_CH, cols), data.dtype),
            pltpu.SemaphoreType.DMA,
            pltpu.SemaphoreType.DMA,
            pltpu.SemaphoreType.DMA,
            pltpu.SemaphoreType.DMA,
        ],
    )
    def sc_copy(x_hbm, o_hbm, b0, b1, si0, si1, so0, so1):
        wid = lax.axis_index("s") * _NC + lax.axis_index("c")
        base = wid * rows_per_w
        bufs = (b0, b1)
        in_sems = (si0, si1)
        out_sems = (so0, so1)

        def in_copy(j):
            return pltpu.async_copy(
                x_hbm.at[pl.ds(base + j * _CH, _CH)], bufs[j % 2], in_sems[j % 2])

        def out_copy(j):
            return pltpu.async_copy(
                bufs[j % 2], o_hbm.at[pl.ds(base + j * _CH, _CH)], out_sems[j % 2])

        pending_in = {0: in_copy(0)}
        pending_out = {}
        for j in range(n_chunks):
            pending_in.pop(j).wait()
            pending_out[j] = out_copy(j)
            if j + 1 < n_chunks:
                if j - 1 in pending_out:
                    pending_out.pop(j - 1).wait()
                pending_in[j + 1] = in_copy(j + 1)
        for j in list(pending_out):
            pending_out.pop(j).wait()

    return sc_copy(data)


# speedup vs baseline: 1.3969x; 1.3969x over previous
import jax
import jax.numpy as jnp
from jax.experimental import pallas as pl
from jax.experimental.pallas import tpu as pltpu

_BLOCK_ROWS = 2048


def _copy_block(x_ref, o_ref):
    o_ref[...] = x_ref[...]


def kernel(data, partitions):
    del partitions
    rows, cols = data.shape
    return pl.pallas_call(
        _copy_block,
        grid=(rows // _BLOCK_ROWS,),
        in_specs=[pl.BlockSpec((_BLOCK_ROWS, cols), lambda i: (i, 0))],
        out_specs=pl.BlockSpec((_BLOCK_ROWS, cols), lambda i: (i, 0)),
        out_shape=jax.ShapeDtypeStruct((rows, cols), data.dtype),
        compiler_params=pltpu.CompilerParams(
            dimension_semantics=("parallel",),
            vmem_limit_bytes=100 * 1024 * 1024,
        ),
    )(data)
